# Initial kernel scaffold; baseline (speedup 1.0000x reference)
import jax
import jax.numpy as jnp
from jax.experimental import pallas as pl


def kernel(x, W, b, centroids):
    raise NotImplementedError("write your pallas kernel here")


# speedup vs baseline: 1.9725x; 1.9725x over previous
import jax
import jax.numpy as jnp
from jax import lax
from jax.experimental import pallas as pl
from jax.experimental.pallas import tpu as pltpu

_N_VARS = 512
_N_CLUSTER = 64
_SEQ_LEN = 4096
_HIDDEN = 1024
_BATCH = 32
_SEQ_BLK = 1024


def _mean_kernel(x_ref, out_ref):
    bidx = pl.program_id(0)
    xb = x_ref[0].astype(jnp.bfloat16).astype(jnp.float32)

    @pl.when(bidx == 0)
    def _init():
        out_ref[...] = xb

    @pl.when(bidx > 0)
    def _accum():
        out_ref[...] += xb


def _bf16_dot(wb, xm):
    hi = xm.astype(jnp.bfloat16)
    r1 = xm - hi.astype(jnp.float32)
    lo = r1.astype(jnp.bfloat16)
    r2 = r1 - lo.astype(jnp.float32)
    lo2 = r2.astype(jnp.bfloat16)
    dims = (((1,), (0,)), ((), ()))
    acc = lax.dot_general(wb, hi, dims, preferred_element_type=jnp.float32)
    acc += lax.dot_general(wb, lo, dims, preferred_element_type=jnp.float32)
    acc += lax.dot_general(wb, lo2, dims, preferred_element_type=jnp.float32)
    return acc


def _head_kernel(w_ref, xm_ref, b_ref, c_ref, out_ref, et_ref):
    sidx = pl.program_id(0)
    ns = pl.num_programs(0)

    part = _bf16_dot(w_ref[...], xm_ref[...])

    @pl.when(sidx == 0)
    def _init():
        et_ref[...] = part

    @pl.when(sidx > 0)
    def _accum():
        et_ref[...] += part

    @pl.when(sidx == ns - 1)
    def _head():
        et = et_ref[...] * (1.0 / _BATCH) + b_ref[...]
        norm = jnp.sqrt(jnp.sum(et * et, axis=0, keepdims=True))
        en = (et / jnp.maximum(norm, 1e-12)).astype(jnp.bfloat16)
        c = c_ref[...]
        cnorm = jnp.sqrt(jnp.sum(c * c, axis=1, keepdims=True))
        cn = (c / jnp.maximum(cnorm, 1e-12)).astype(jnp.bfloat16)
        st = lax.dot_general(
            cn, en,
            dimension_numbers=(((1,), (0,)), ((), ())),
            preferred_element_type=jnp.float32,
        )
        m = jnp.max(st, axis=0, keepdims=True)
        iota_k = lax.broadcasted_iota(jnp.int32, (_N_CLUSTER, _N_VARS), 0)
        masked = jnp.where(st >= m, iota_k, _N_CLUSTER)
        idx = jnp.min(masked, axis=0, keepdims=True)
        pt = (iota_k == idx).astype(jnp.float32)
        r = lax.broadcasted_iota(jnp.int32, (_N_VARS, _N_VARS), 0)
        q = lax.broadcasted_iota(jnp.int32, (_N_VARS, _N_VARS), 1)
        eye = (r == q).astype(jnp.float32)
        out_ref[...] = lax.dot_general(
            eye, pt,
            dimension_numbers=(((1,), (1,)), ((), ())),
            preferred_element_type=jnp.float32,
        )


def kernel(x, W, b, centroids):
    xsum = pl.pallas_call(
        _mean_kernel,
        grid=(_BATCH,),
        in_specs=[pl.BlockSpec((1, _SEQ_LEN, _N_VARS), lambda i: (i, 0, 0))],
        out_specs=pl.BlockSpec((_SEQ_LEN, _N_VARS), lambda i: (0, 0)),
        out_shape=jax.ShapeDtypeStruct((_SEQ_LEN, _N_VARS), jnp.float32),
    )(x)

    wb16 = W.astype(jnp.bfloat16)
    b2 = b.reshape(_HIDDEN, 1)
    return pl.pallas_call(
        _head_kernel,
        grid=(_SEQ_LEN // _SEQ_BLK,),
        in_specs=[
            pl.BlockSpec((_HIDDEN, _SEQ_BLK), lambda i: (0, i)),
            pl.BlockSpec((_SEQ_BLK, _N_VARS), lambda i: (i, 0)),
            pl.BlockSpec((_HIDDEN, 1), lambda i: (0, 0)),
            pl.BlockSpec((_N_CLUSTER, _HIDDEN), lambda i: (0, 0)),
        ],
        out_specs=pl.BlockSpec((_N_VARS, _N_CLUSTER), lambda i: (0, 0)),
        out_shape=jax.ShapeDtypeStruct((_N_VARS, _N_CLUSTER), jnp.float32),
        scratch_shapes=[pltpu.VMEM((_HIDDEN, _N_VARS), jnp.float32)],
    )(wb16, xsum, b2, centroids)


# speedup vs baseline: 2.0636x; 1.0462x over previous
import jax
import jax.numpy as jnp
from jax import lax
from jax.experimental import pallas as pl
from jax.experimental.pallas import tpu as pltpu

_N_VARS = 512
_N_CLUSTER = 64
_SEQ_LEN = 4096
_HIDDEN = 1024
_BATCH = 32
_SEQ_BLK = 1024


def _bf16_dot(wb, xm):
    hi = xm.astype(jnp.bfloat16)
    r1 = xm - hi.astype(jnp.float32)
    lo = r1.astype(jnp.bfloat16)
    r2 = r1 - lo.astype(jnp.float32)
    lo2 = r2.astype(jnp.bfloat16)
    dims = (((1,), (0,)), ((), ()))
    acc = lax.dot_general(wb, hi, dims, preferred_element_type=jnp.float32)
    acc += lax.dot_general(wb, lo, dims, preferred_element_type=jnp.float32)
    acc += lax.dot_general(wb, lo2, dims, preferred_element_type=jnp.float32)
    return acc


def _fused_kernel(x_ref, w_ref, b_ref, c_ref, out_ref, acc_ref, et_ref):
    bidx = pl.program_id(0)
    xb = x_ref[0].astype(jnp.bfloat16).astype(jnp.float32)

    @pl.when(bidx == 0)
    def _init():
        acc_ref[...] = xb

    @pl.when(bidx > 0)
    def _accum():
        acc_ref[...] += xb

    @pl.when(bidx == _BATCH - 1)
    def _head():
        for i in range(_SEQ_LEN // _SEQ_BLK):
            sl = slice(i * _SEQ_BLK, (i + 1) * _SEQ_BLK)
            part = _bf16_dot(w_ref[:, sl], acc_ref[sl, :])
            if i == 0:
                et_ref[...] = part
            else:
                et_ref[...] += part
        et = et_ref[...] * (1.0 / _BATCH) + b_ref[...]
        norm = jnp.sqrt(jnp.sum(et * et, axis=0, keepdims=True))
        en = (et / jnp.maximum(norm, 1e-12)).astype(jnp.bfloat16)
        c = c_ref[...]
        cnorm = jnp.sqrt(jnp.sum(c * c, axis=1, keepdims=True))
        cn = (c / jnp.maximum(cnorm, 1e-12)).astype(jnp.bfloat16)
        st = lax.dot_general(
            cn, en,
            dimension_numbers=(((1,), (0,)), ((), ())),
            preferred_element_type=jnp.float32,
        )
        m = jnp.max(st, axis=0, keepdims=True)
        iota_k = lax.broadcasted_iota(jnp.int32, (_N_CLUSTER, _N_VARS), 0)
        masked = jnp.where(st >= m, iota_k, _N_CLUSTER)
        idx = jnp.min(masked, axis=0, keepdims=True)
        pt = (iota_k == idx).astype(jnp.float32)
        r = lax.broadcasted_iota(jnp.int32, (_N_VARS, _N_VARS), 0)
        q = lax.broadcasted_iota(jnp.int32, (_N_VARS, _N_VARS), 1)
        eye = (r == q).astype(jnp.float32)
        out_ref[...] = lax.dot_general(
            eye, pt,
            dimension_numbers=(((1,), (1,)), ((), ())),
            preferred_element_type=jnp.float32,
        )


def kernel(x, W, b, centroids):
    wb16 = W.astype(jnp.bfloat16)
    b2 = b.reshape(_HIDDEN, 1)
    return pl.pallas_call(
        _fused_kernel,
        grid=(_BATCH,),
        in_specs=[
            pl.BlockSpec((1, _SEQ_LEN, _N_VARS), lambda i: (i, 0, 0)),
            pl.BlockSpec((_HIDDEN, _SEQ_LEN), lambda i: (0, 0)),
            pl.BlockSpec((_HIDDEN, 1), lambda i: (0, 0)),
            pl.BlockSpec((_N_CLUSTER, _HIDDEN), lambda i: (0, 0)),
        ],
        out_specs=pl.BlockSpec((_N_VARS, _N_CLUSTER), lambda i: (0, 0)),
        out_shape=jax.ShapeDtypeStruct((_N_VARS, _N_CLUSTER), jnp.float32),
        scratch_shapes=[
            pltpu.VMEM((_SEQ_LEN, _N_VARS), jnp.float32),
            pltpu.VMEM((_HIDDEN, _N_VARS), jnp.float32),
        ],
    )(x, wb16, b2, centroids)


# speedup vs baseline: 2.0656x; 1.0009x over previous
import jax
import jax.numpy as jnp
from jax import lax
from jax.experimental import pallas as pl
from jax.experimental.pallas import tpu as pltpu

_N_VARS = 512
_N_CLUSTER = 64
_SEQ_LEN = 4096
_HIDDEN = 1024
_BATCH = 32
_SEQ_BLK = 2048


def _bf16_dot(wb, xm):
    hi = xm.astype(jnp.bfloat16)
    r1 = xm - hi.astype(jnp.float32)
    lo = r1.astype(jnp.bfloat16)
    r2 = r1 - lo.astype(jnp.float32)
    lo2 = r2.astype(jnp.bfloat16)
    dims = (((1,), (0,)), ((), ()))
    acc = lax.dot_general(wb, hi, dims, preferred_element_type=jnp.float32)
    acc += lax.dot_general(wb, lo, dims, preferred_element_type=jnp.float32)
    acc += lax.dot_general(wb, lo2, dims, preferred_element_type=jnp.float32)
    return acc


def _fused_kernel(x_ref, w_ref, b_ref, c_ref, out_ref, acc_ref, et_ref):
    bidx = pl.program_id(0)
    xb = x_ref[0].astype(jnp.bfloat16).astype(jnp.float32)

    @pl.when(bidx == 0)
    def _init():
        acc_ref[...] = xb

    @pl.when(bidx > 0)
    def _accum():
        acc_ref[...] += xb

    @pl.when(bidx == _BATCH - 1)
    def _head():
        for i in range(_SEQ_LEN // _SEQ_BLK):
            sl = slice(i * _SEQ_BLK, (i + 1) * _SEQ_BLK)
            part = _bf16_dot(w_ref[:, sl], acc_ref[sl, :])
            if i == 0:
                et_ref[...] = part
            else:
                et_ref[...] += part
        et = et_ref[...] * (1.0 / _BATCH) + b_ref[...]
        norm = jnp.sqrt(jnp.sum(et * et, axis=0, keepdims=True))
        en = (et / jnp.maximum(norm, 1e-12)).astype(jnp.bfloat16)
        c = c_ref[...]
        cnorm = jnp.sqrt(jnp.sum(c * c, axis=1, keepdims=True))
        cn = (c / jnp.maximum(cnorm, 1e-12)).astype(jnp.bfloat16)
        st = lax.dot_general(
            cn, en,
            dimension_numbers=(((1,), (0,)), ((), ())),
            preferred_element_type=jnp.float32,
        )
        m = jnp.max(st, axis=0, keepdims=True)
        iota_k = lax.broadcasted_iota(jnp.int32, (_N_CLUSTER, _N_VARS), 0)
        masked = jnp.where(st >= m, iota_k, _N_CLUSTER)
        idx = jnp.min(masked, axis=0, keepdims=True)
        pt = (iota_k == idx).astype(jnp.float32)
        r = lax.broadcasted_iota(jnp.int32, (_N_VARS, _N_VARS), 0)
        q = lax.broadcasted_iota(jnp.int32, (_N_VARS, _N_VARS), 1)
        eye = (r == q).astype(jnp.float32)
        out_ref[...] = lax.dot_general(
            eye, pt,
            dimension_numbers=(((1,), (1,)), ((), ())),
            preferred_element_type=jnp.float32,
        )


def kernel(x, W, b, centroids):
    wb16 = W.astype(jnp.bfloat16)
    b2 = b.reshape(_HIDDEN, 1)
    return pl.pallas_call(
        _fused_kernel,
        grid=(_BATCH,),
        in_specs=[
            pl.BlockSpec((1, _SEQ_LEN, _N_VARS), lambda i: (i, 0, 0)),
            pl.BlockSpec((_HIDDEN, _SEQ_LEN), lambda i: (0, 0)),
            pl.BlockSpec((_HIDDEN, 1), lambda i: (0, 0)),
            pl.BlockSpec((_N_CLUSTER, _HIDDEN), lambda i: (0, 0)),
        ],
        out_specs=pl.BlockSpec((_N_VARS, _N_CLUSTER), lambda i: (0, 0)),
        out_shape=jax.ShapeDtypeStruct((_N_VARS, _N_CLUSTER), jnp.float32),
        scratch_shapes=[
            pltpu.VMEM((_SEQ_LEN, _N_VARS), jnp.float32),
            pltpu.VMEM((_HIDDEN, _N_VARS), jnp.float32),
        ],
    )(x, wb16, b2, centroids)
